# baseline (speedup 1.0000x reference)
import functools

import jax
import jax.numpy as jnp
from jax import lax
from jax.experimental import pallas as pl
from jax.experimental.pallas import tpu as pltpu
from jax.experimental.pallas import tpu_sc as plsc

B = 16384
D = 128
NUM_REL = 1000
LANES = 16
NGROUPS = D // LANES
CHUNK = 64
EPS = 1e-12


def _rsqrt_nr(x):
    i = lax.bitcast_convert_type(x, jnp.int32)
    i = jnp.int32(0x5F3759DF) - lax.shift_right_logical(i, 1)
    y = lax.bitcast_convert_type(i, jnp.float32)
    for _ in range(2):
        y = y * (1.5 - 0.5 * x * y * y)
    return y


def _sqrt_via_rsqrt(x):
    return x * _rsqrt_nr(x)


def _make_kernel():
    nc, ns = 2, 16
    nw = nc * ns
    bpw = B // nw
    nchunks = bpw // CHUNK
    mesh = plsc.VectorSubcoreMesh(
        core_axis_name="c", subcore_axis_name="s", num_cores=nc, num_subcores=ns
    )

    @functools.partial(
        pl.kernel,
        out_type=jax.ShapeDtypeStruct((B,), jnp.float32),
        mesh=mesh,
        compiler_params=pltpu.CompilerParams(needs_layout_passes=False),
        scratch_types=[
            pltpu.VMEM((bpw,), jnp.int32),
            pltpu.VMEM((bpw,), jnp.int32),
            pltpu.VMEM((bpw,), jnp.int32),
            pltpu.VMEM((4, CHUNK, D), jnp.float32),
            pltpu.VMEM((4, CHUNK, D), jnp.float32),
            pltpu.VMEM((4, CHUNK, D), jnp.float32),
            pltpu.VMEM((bpw,), jnp.float32),
            pltpu.VMEM((6, LANES), jnp.float32),
            pltpu.SemaphoreType.DMA,
            pltpu.SemaphoreType.DMA,
            pltpu.SemaphoreType.DMA,
            pltpu.SemaphoreType.DMA,
        ],
    )
    def kern(head_hbm, rel_hbm, tail_hbm, ent_hbm, remb_hbm, out_hbm,
             hidx_v, ridx_v, tidx_v, hrows_v, rrows_v, trows_v, out_v,
             stage_v, sem0, sem1, sem2, sem3):
        wid = lax.axis_index("s") * nc + lax.axis_index("c")
        base = wid * bpw
        pltpu.sync_copy(head_hbm.at[pl.ds(base, bpw)], hidx_v)
        pltpu.sync_copy(rel_hbm.at[pl.ds(base, bpw)], ridx_v)
        pltpu.sync_copy(tail_hbm.at[pl.ds(base, bpw)], tidx_v)

        sems = (sem0, sem1, sem2, sem3)
        tabs = (ent_hbm, remb_hbm, ent_hbm)
        idxs = (hidx_v, ridx_v, tidx_v)
        rows = (hrows_v, rrows_v, trows_v)

        def descs(c, buf):
            sl = pl.ds(c * CHUNK, CHUNK)
            return [
                pltpu.make_async_copy(
                    tabs[k].at[idxs[k].at[sl]], rows[k].at[buf], sems[buf]
                )
                for k in range(3)
            ]

        def start(c, buf):
            for d in descs(c, buf):
                d.start()

        def wait(c, buf):
            for d in descs(c, buf):
                d.wait()

        def compute(c, buf):
            lane15 = lax.iota(jnp.int32, LANES) == (LANES - 1)
            posjs = [jnp.full((LANES,), j, jnp.int32) for j in range(LANES)]
            qvs = [jnp.full((LANES,), q, jnp.int32) for q in range(6)]

            def group_body(g, carry):
                for j in range(LANES):
                    i = g * LANES + j
                    hs = [hrows_v[buf, i, pl.ds(k * LANES, LANES)] for k in range(NGROUPS)]
                    ts = [trows_v[buf, i, pl.ds(k * LANES, LANES)] for k in range(NGROUPS)]
                    rs = [rrows_v[buf, i, pl.ds(k * LANES, LANES)] for k in range(NGROUPS)]
                    acc = [None] * 6
                    for k in range(NGROUPS):
                        prods = (
                            hs[k] * hs[k], ts[k] * ts[k], rs[k] * rs[k],
                            hs[k] * rs[k], hs[k] * ts[k], rs[k] * ts[k],
                        )
                        for q in range(6):
                            acc[q] = prods[q] if k == 0 else acc[q] + prods[q]
                    for q in range(6):
                        plsc.store_scatter(
                            stage_v, [qvs[q], posjs[j]], plsc.cumsum(acc[q]), mask=lane15
                        )
                ssh = stage_v[0, :]
                sst = stage_v[1, :]
                srr = stage_v[2, :]
                shr = stage_v[3, :]
                sht = stage_v[4, :]
                srt = stage_v[5, :]
                inv_h = jnp.minimum(_rsqrt_nr(ssh), 1.0 / EPS)
                inv_t = jnp.minimum(_rsqrt_nr(sst), 1.0 / EPS)
                ssc = (
                    ssh * inv_h * inv_h
                    + srr
                    + sst * inv_t * inv_t
                    + 2.0 * (inv_h * shr - inv_h * inv_t * sht - inv_t * srt)
                )
                ssc = jnp.maximum(ssc, 0.0)
                out_v[pl.ds(c * CHUNK + g * LANES, LANES)] = _sqrt_via_rsqrt(ssc)
                return carry

            lax.fori_loop(0, CHUNK // LANES, group_body, 0)

        for b in range(3):
            start(b, b)

        def quad(q, carry):
            c0 = 4 * q
            for k in range(4):
                c = c0 + k
                nxt = c + 3

                @pl.when(nxt < nchunks)
                def _(nxt=nxt, k=k):
                    start(nxt, (k + 3) % 4)

                wait(c, k)
                compute(c, k)
            return carry

        lax.fori_loop(0, nchunks // 4, quad, 0)

        pltpu.sync_copy(out_v, out_hbm.at[pl.ds(base, bpw)])

    return kern


@functools.cache
def _get_kernel():
    return _make_kernel()


def kernel(head_index, rel_index, tail_index, ent_emb, rel_emb):
    return _get_kernel()(
        head_index.astype(jnp.int32),
        rel_index.astype(jnp.int32),
        tail_index.astype(jnp.int32),
        ent_emb,
        rel_emb,
    )


# speedup vs baseline: 1.1335x; 1.1335x over previous
import functools

import jax
import jax.numpy as jnp
from jax import lax
from jax.experimental import pallas as pl
from jax.experimental.pallas import tpu as pltpu
from jax.experimental.pallas import tpu_sc as plsc

B = 16384
D = 128
NUM_REL = 1000
LANES = 16
NGROUPS = D // LANES
CHUNK = 64
EPS = 1e-12


def _rsqrt_nr(x):
    i = lax.bitcast_convert_type(x, jnp.int32)
    i = jnp.int32(0x5F3759DF) - lax.shift_right_logical(i, 1)
    y = lax.bitcast_convert_type(i, jnp.float32)
    for _ in range(2):
        y = y * (1.5 - 0.5 * x * y * y)
    return y


def _sqrt_via_rsqrt(x):
    return x * _rsqrt_nr(x)


def _make_kernel():
    nc, ns = 2, 16
    nw = nc * ns
    bpw = B // nw
    nchunks = bpw // CHUNK
    mesh = plsc.VectorSubcoreMesh(
        core_axis_name="c", subcore_axis_name="s", num_cores=nc, num_subcores=ns
    )

    @functools.partial(
        pl.kernel,
        out_type=jax.ShapeDtypeStruct((B,), jnp.float32),
        mesh=mesh,
        compiler_params=pltpu.CompilerParams(needs_layout_passes=False),
        scratch_types=[
            pltpu.VMEM((bpw,), jnp.int32),
            pltpu.VMEM((bpw,), jnp.int32),
            pltpu.VMEM((bpw,), jnp.int32),
            pltpu.VMEM((2, CHUNK, D), jnp.float32),
            pltpu.VMEM((2, CHUNK, D), jnp.float32),
            pltpu.VMEM((2, CHUNK, D), jnp.float32),
            pltpu.VMEM((bpw,), jnp.float32),
            pltpu.VMEM((6, LANES), jnp.float32),
            pltpu.SemaphoreType.DMA,
            pltpu.SemaphoreType.DMA,
        ],
    )
    def kern(head_hbm, rel_hbm, tail_hbm, ent_hbm, remb_hbm, out_hbm,
             hidx_v, ridx_v, tidx_v, hrows_v, rrows_v, trows_v, out_v,
             stage_v, sem0, sem1):
        wid = lax.axis_index("s") * nc + lax.axis_index("c")
        base = wid * bpw
        pltpu.sync_copy(head_hbm.at[pl.ds(base, bpw)], hidx_v)
        pltpu.sync_copy(rel_hbm.at[pl.ds(base, bpw)], ridx_v)
        pltpu.sync_copy(tail_hbm.at[pl.ds(base, bpw)], tidx_v)

        sems = (sem0, sem1)
        tabs = (ent_hbm, remb_hbm, ent_hbm)
        idxs = (hidx_v, ridx_v, tidx_v)
        rows = (hrows_v, rrows_v, trows_v)

        def descs(c, buf):
            sl = pl.ds(c * CHUNK, CHUNK)
            return [
                pltpu.make_async_copy(
                    tabs[k].at[idxs[k].at[sl]], rows[k].at[buf], sems[buf]
                )
                for k in range(3)
            ]

        def start(c, buf):
            for d in descs(c, buf):
                d.start()

        def wait(c, buf):
            for d in descs(c, buf):
                d.wait()

        def compute(c, buf):
            lane15 = lax.iota(jnp.int32, LANES) == (LANES - 1)
            posjs = [jnp.full((LANES,), j, jnp.int32) for j in range(LANES)]
            qvs = [jnp.full((LANES,), q, jnp.int32) for q in range(6)]

            def group_body(g, carry):
                for j in range(LANES):
                    i = g * LANES + j
                    hs = [hrows_v[buf, i, pl.ds(k * LANES, LANES)] for k in range(NGROUPS)]
                    ts = [trows_v[buf, i, pl.ds(k * LANES, LANES)] for k in range(NGROUPS)]
                    rs = [rrows_v[buf, i, pl.ds(k * LANES, LANES)] for k in range(NGROUPS)]
                    acc = [None] * 6
                    for k in range(NGROUPS):
                        prods = (
                            hs[k] * hs[k], ts[k] * ts[k], rs[k] * rs[k],
                            hs[k] * rs[k], hs[k] * ts[k], rs[k] * ts[k],
                        )
                        for q in range(6):
                            acc[q] = prods[q] if k == 0 else acc[q] + prods[q]
                    for q in range(6):
                        plsc.store_scatter(
                            stage_v, [qvs[q], posjs[j]], plsc.cumsum(acc[q]), mask=lane15
                        )
                ssh = stage_v[0, :]
                sst = stage_v[1, :]
                srr = stage_v[2, :]
                shr = stage_v[3, :]
                sht = stage_v[4, :]
                srt = stage_v[5, :]
                inv_h = jnp.minimum(_rsqrt_nr(ssh), 1.0 / EPS)
                inv_t = jnp.minimum(_rsqrt_nr(sst), 1.0 / EPS)
                ssc = (
                    ssh * inv_h * inv_h
                    + srr
                    + sst * inv_t * inv_t
                    + 2.0 * (inv_h * shr - inv_h * inv_t * sht - inv_t * srt)
                )
                ssc = jnp.maximum(ssc, 0.0)
                out_v[pl.ds(c * CHUNK + g * LANES, LANES)] = _sqrt_via_rsqrt(ssc)
                return carry

            lax.fori_loop(0, CHUNK // LANES, group_body, 0)

        start(0, 0)

        def pair(p, carry):
            c0 = 2 * p
            start(c0 + 1, 1)
            wait(c0, 0)
            compute(c0, 0)

            @pl.when(c0 + 2 < nchunks)
            def _():
                start(c0 + 2, 0)

            wait(c0 + 1, 1)
            compute(c0 + 1, 1)
            return carry

        lax.fori_loop(0, nchunks // 2, pair, 0)

        pltpu.sync_copy(out_v, out_hbm.at[pl.ds(base, bpw)])

    return kern


@functools.cache
def _get_kernel():
    return _make_kernel()


def kernel(head_index, rel_index, tail_index, ent_emb, rel_emb):
    return _get_kernel()(
        head_index.astype(jnp.int32),
        rel_index.astype(jnp.int32),
        tail_index.astype(jnp.int32),
        ent_emb,
        rel_emb,
    )
